# baseline (speedup 1.0000x reference)
import functools

import jax
import jax.numpy as jnp
from jax import lax
from jax.experimental import pallas as pl
from jax.experimental.pallas import tpu as pltpu
from jax.experimental.pallas import tpu_sc as plsc

N_NODES = 10000
D_IN = 128
D_H = 16
D_OUT = 7

NC = 2
NS = 16
NW = NC * NS

N_PAD = 10240
CHUNK = 128
ROWS_PER_TILE = N_PAD // NS
L = 16


def _aggregate(d, kj, eb_hbm, out_hbm, src_v, dst_v, rows_v, xs, acc, sem, sem_s,
               cid, sid, wid):
    pltpu.sync_copy(eb_hbm.at[0, wid], src_v)
    pltpu.sync_copy(eb_hbm.at[1, wid], dst_v)
    plsc.subcore_barrier()

    pltpu.async_copy(xs.at[src_v.at[0]], rows_v.at[0], sem)
    pltpu.async_copy(xs.at[src_v.at[1]], rows_v.at[1], sem)

    def _chunk(j, carry):
        buf = lax.rem(j, 4)
        pltpu.make_async_copy(xs.at[src_v.at[j]], rows_v.at[buf], sem).wait()

        @pl.when(j >= 2)
        def _():
            b2 = lax.rem(j + 2, 4)
            pltpu.make_async_copy(
                rows_v.at[b2], acc.at[dst_v.at[j - 2]], sem_s
            ).wait()

        @pl.when(j + 2 < kj)
        def _():
            pltpu.async_copy(
                xs.at[src_v.at[j + 2]], rows_v.at[lax.rem(j + 2, 4)], sem
            )

        pltpu.async_copy(rows_v.at[buf], acc.at[dst_v.at[j]], sem_s, add=True)
        return carry

    lax.fori_loop(0, kj, _chunk, 0, unroll=4)
    for j in (kj - 2, kj - 1):
        pltpu.make_async_copy(
            rows_v.at[lax.rem(j, 4)], acc.at[dst_v.at[j]], sem_s
        ).wait()
    plsc.subcore_barrier()

    pltpu.sync_copy(
        acc.at[pl.ds(sid * ROWS_PER_TILE, ROWS_PER_TILE)],
        out_hbm.at[cid, pl.ds(sid * ROWS_PER_TILE, ROWS_PER_TILE)],
    )


def _zero_acc(d, zbuf, acc, sid):
    if d == L:
        @plsc.parallel_loop(0, ROWS_PER_TILE, unroll=4)
        def _zero(i):
            zbuf[i, :] = jnp.zeros((L,), jnp.float32)
    else:
        zf = jnp.zeros((L,), jnp.float32)
        zi = jnp.zeros((L,), jnp.int32)
        riota = lax.iota(jnp.int32, L)

        @plsc.parallel_loop(0, ROWS_PER_TILE // L, unroll=2)
        def _zero(g):
            rows = g * L + riota
            for j in range(d):
                plsc.store_scatter(zbuf, [rows, zi + j], zf)
    pltpu.sync_copy(zbuf, acc.at[pl.ds(sid * ROWS_PER_TILE, ROWS_PER_TILE)])


def _spmm1_body(kj, x_hbm, eb_hbm, out_hbm,
                src_v, dst_v, rows_v, zbuf, xs, acc, sem, sem_s):
    cid = lax.axis_index("c")
    sid = lax.axis_index("s")
    wid = sid * NC + cid

    pltpu.sync_copy(
        x_hbm.at[pl.ds(sid * ROWS_PER_TILE, ROWS_PER_TILE)],
        xs.at[pl.ds(sid * ROWS_PER_TILE, ROWS_PER_TILE)],
    )
    _zero_acc(D_H, zbuf, acc, sid)
    _aggregate(D_H, kj, eb_hbm, out_hbm, src_v, dst_v, rows_v, xs, acc, sem, sem_s,
               cid, sid, wid)


def _spmm2_body(kj, p_hbm, eb_hbm, w_hbm, out_hbm,
                src_v, dst_v, rows_v, zbuf, pv, x2v, w_v, xs, acc, sem, sem_s):
    cid = lax.axis_index("c")
    sid = lax.axis_index("s")
    wid = sid * NC + cid
    base = sid * ROWS_PER_TILE

    pltpu.sync_copy(w_hbm, w_v)
    pltpu.sync_copy(p_hbm.at[0, pl.ds(base, ROWS_PER_TILE)], pv.at[0])
    pltpu.sync_copy(p_hbm.at[1, pl.ds(base, ROWS_PER_TILE)], pv.at[1])

    zero_i = jnp.zeros((L,), jnp.int32)
    riota = lax.iota(jnp.int32, L)
    wregs = [w_v[pl.ds(t * L, L)] for t in range(8 * D_H // L)]

    def _wscal(k, j):
        flat = k * 8 + j
        return wregs[flat // L][flat % L]

    def _group(g, carry):
        rows = g * L + riota
        cols = [None] * D_H
        for k in range(D_H):
            a = plsc.load_gather(pv, [zero_i, rows, zero_i + k])
            b = plsc.load_gather(pv, [zero_i + 1, rows, zero_i + k])
            cols[k] = jnp.maximum(a + b, 0.0)
        for j in range(8):
            o = cols[0] * _wscal(0, j)
            for k in range(1, D_H):
                o = o + cols[k] * _wscal(k, j)
            plsc.store_scatter(x2v, [rows, zero_i + j], o)
        return carry

    lax.fori_loop(0, ROWS_PER_TILE // L, _group, 0, unroll=2)
    pltpu.sync_copy(x2v, xs.at[pl.ds(base, ROWS_PER_TILE)])

    _zero_acc(8, zbuf, acc, sid)
    _aggregate(8, kj, eb_hbm, out_hbm, src_v, dst_v, rows_v, xs, acc, sem, sem_s,
               cid, sid, wid)


@functools.cache
def _build_spmm1(kj):
    mesh = plsc.VectorSubcoreMesh(
        core_axis_name="c", subcore_axis_name="s", num_cores=NC, num_subcores=NS
    )
    return pl.kernel(
        functools.partial(_spmm1_body, kj),
        out_type=jax.ShapeDtypeStruct((NC, N_PAD, D_H), jnp.float32),
        mesh=mesh,
        scratch_types=[
            pltpu.VMEM((kj, CHUNK), jnp.int32),
            pltpu.VMEM((kj, CHUNK), jnp.int32),
            pltpu.VMEM((4, CHUNK, D_H), jnp.float32),
            pltpu.VMEM((ROWS_PER_TILE, D_H), jnp.float32),
            pltpu.VMEM_SHARED((N_PAD, D_H), jnp.float32),
            pltpu.VMEM_SHARED((N_PAD, D_H), jnp.float32),
            pltpu.SemaphoreType.DMA,
            pltpu.SemaphoreType.DMA,
        ],
        compiler_params=pltpu.CompilerParams(use_tc_tiling_on_sc=False),
    )


@functools.cache
def _build_spmm2(kj):
    mesh = plsc.VectorSubcoreMesh(
        core_axis_name="c", subcore_axis_name="s", num_cores=NC, num_subcores=NS
    )
    return pl.kernel(
        functools.partial(_spmm2_body, kj),
        out_type=jax.ShapeDtypeStruct((NC, N_PAD, 8), jnp.float32),
        mesh=mesh,
        scratch_types=[
            pltpu.VMEM((kj, CHUNK), jnp.int32),
            pltpu.VMEM((kj, CHUNK), jnp.int32),
            pltpu.VMEM((4, CHUNK, 8), jnp.float32),
            pltpu.VMEM((ROWS_PER_TILE, 8), jnp.float32),
            pltpu.VMEM((2, ROWS_PER_TILE, D_H), jnp.float32),
            pltpu.VMEM((ROWS_PER_TILE, 8), jnp.float32),
            pltpu.VMEM((8 * D_H,), jnp.float32),
            pltpu.VMEM_SHARED((N_PAD, 8), jnp.float32),
            pltpu.VMEM_SHARED((N_PAD, 8), jnp.float32),
            pltpu.SemaphoreType.DMA,
            pltpu.SemaphoreType.DMA,
        ],
        compiler_params=pltpu.CompilerParams(
            use_tc_tiling_on_sc=False, needs_layout_passes=False
        ),
    )


ROWS_PER_WORKER = 400
N_COMBINE_W = N_NODES // ROWS_PER_WORKER


def _combine_sc_body(q_hbm, out_hbm, qv, sv):
    cid = lax.axis_index("c")
    sid = lax.axis_index("s")
    wid = sid * NC + cid

    @pl.when(wid < N_COMBINE_W)
    def _():
        base = wid * ROWS_PER_WORKER
        pltpu.sync_copy(q_hbm.at[0, pl.ds(base, ROWS_PER_WORKER)], qv.at[0])
        pltpu.sync_copy(q_hbm.at[1, pl.ds(base, ROWS_PER_WORKER)], qv.at[1])

        zero_i = jnp.zeros((L,), jnp.int32)
        lane = lax.iota(jnp.int32, L)

        @plsc.parallel_loop(0, ROWS_PER_WORKER * D_OUT // L, unroll=2)
        def _pack(t):
            fo = t * L + lane
            r = fo // D_OUT
            c = fo - r * D_OUT
            a = plsc.load_gather(qv, [zero_i, r, c])
            b = plsc.load_gather(qv, [zero_i + 1, r, c])
            plsc.store_scatter(sv, [r, c], a + b)

        pltpu.sync_copy(sv, out_hbm.at[pl.ds(base, ROWS_PER_WORKER)])


@functools.cache
def _build_combine_sc():
    mesh = plsc.VectorSubcoreMesh(
        core_axis_name="c", subcore_axis_name="s", num_cores=NC, num_subcores=NS
    )
    return pl.kernel(
        _combine_sc_body,
        out_type=jax.ShapeDtypeStruct((N_NODES, D_OUT), jnp.float32),
        mesh=mesh,
        scratch_types=[
            pltpu.VMEM((2, ROWS_PER_WORKER, 8), jnp.float32),
            pltpu.VMEM((ROWS_PER_WORKER, D_OUT), jnp.float32),
        ],
        compiler_params=pltpu.CompilerParams(
            use_tc_tiling_on_sc=False, needs_layout_passes=False
        ),
    )


def _mm1_body(f_ref, w1_ref, o_ref):
    o_ref[...] = jnp.dot(f_ref[...], w1_ref[...],
                         preferred_element_type=jnp.float32)


def kernel(features, edge_index, W1, W2):
    e = edge_index.shape[1]
    kj = (e + NW * CHUNK - 1) // (NW * CHUNK)
    e_pad = NW * CHUNK * kj
    eb = jnp.pad(
        edge_index.astype(jnp.int32),
        ((0, 0), (0, e_pad - e)),
        constant_values=N_NODES,
    ).reshape(2, NW, kj, CHUNK)

    w2f = jnp.pad(W2, ((0, 0), (0, 8 - D_OUT))).reshape(8 * D_H)

    x1 = pl.pallas_call(
        _mm1_body,
        grid=(2,),
        in_specs=[
            pl.BlockSpec((N_PAD // 2, D_IN), lambda i: (i, 0)),
            pl.BlockSpec((D_IN, D_H), lambda i: (0, 0)),
        ],
        out_specs=pl.BlockSpec((N_PAD // 2, D_H), lambda i: (i, 0)),
        out_shape=jax.ShapeDtypeStruct((N_PAD, D_H), jnp.float32),
    )(features, W1)

    p = _build_spmm1(kj)(x1, eb)

    q = _build_spmm2(kj)(p, eb, w2f)

    return _build_combine_sc()(q)


# speedup vs baseline: 1.0152x; 1.0152x over previous
import functools

import jax
import jax.numpy as jnp
from jax import lax
from jax.experimental import pallas as pl
from jax.experimental.pallas import tpu as pltpu
from jax.experimental.pallas import tpu_sc as plsc

N_NODES = 10000
D_IN = 128
D_H = 16
D_OUT = 7

NC = 2
NS = 16
NW = NC * NS

N_PAD = 10240
CHUNK = 128
ROWS_PER_TILE = N_PAD // NS
L = 16


def _aggregate(d, kj, eb_hbm, out_hbm, src_v, dst_v, rows_v, xs, acc, sem, sem_s,
               cid, sid, wid):
    pltpu.sync_copy(eb_hbm.at[0, wid], src_v)
    pltpu.sync_copy(eb_hbm.at[1, wid], dst_v)
    plsc.subcore_barrier()

    pltpu.async_copy(xs.at[src_v.at[0]], rows_v.at[0], sem)
    pltpu.async_copy(xs.at[src_v.at[1]], rows_v.at[1], sem)

    def _chunk(j, carry):
        buf = lax.rem(j, 4)
        pltpu.make_async_copy(xs.at[src_v.at[j]], rows_v.at[buf], sem).wait()

        @pl.when(j >= 2)
        def _():
            b2 = lax.rem(j + 2, 4)
            pltpu.make_async_copy(
                rows_v.at[b2], acc.at[dst_v.at[j - 2]], sem_s
            ).wait()

        @pl.when(j + 2 < kj)
        def _():
            pltpu.async_copy(
                xs.at[src_v.at[j + 2]], rows_v.at[lax.rem(j + 2, 4)], sem
            )

        pltpu.async_copy(rows_v.at[buf], acc.at[dst_v.at[j]], sem_s, add=True)
        return carry

    lax.fori_loop(0, kj, _chunk, 0, unroll=2)
    for j in (kj - 2, kj - 1):
        pltpu.make_async_copy(
            rows_v.at[lax.rem(j, 4)], acc.at[dst_v.at[j]], sem_s
        ).wait()
    plsc.subcore_barrier()

    pltpu.sync_copy(
        acc.at[pl.ds(sid * ROWS_PER_TILE, ROWS_PER_TILE)],
        out_hbm.at[cid, pl.ds(sid * ROWS_PER_TILE, ROWS_PER_TILE)],
    )


def _zero_acc(d, zbuf, acc, sid):
    if d == L:
        @plsc.parallel_loop(0, ROWS_PER_TILE, unroll=4)
        def _zero(i):
            zbuf[i, :] = jnp.zeros((L,), jnp.float32)
    else:
        zf = jnp.zeros((L,), jnp.float32)
        zi = jnp.zeros((L,), jnp.int32)
        riota = lax.iota(jnp.int32, L)

        @plsc.parallel_loop(0, ROWS_PER_TILE // L, unroll=2)
        def _zero(g):
            rows = g * L + riota
            for j in range(d):
                plsc.store_scatter(zbuf, [rows, zi + j], zf)
    pltpu.sync_copy(zbuf, acc.at[pl.ds(sid * ROWS_PER_TILE, ROWS_PER_TILE)])


def _spmm1_body(kj, x_hbm, eb_hbm, out_hbm,
                src_v, dst_v, rows_v, zbuf, xs, acc, sem, sem_s):
    cid = lax.axis_index("c")
    sid = lax.axis_index("s")
    wid = sid * NC + cid

    pltpu.sync_copy(
        x_hbm.at[pl.ds(sid * ROWS_PER_TILE, ROWS_PER_TILE)],
        xs.at[pl.ds(sid * ROWS_PER_TILE, ROWS_PER_TILE)],
    )
    _zero_acc(D_H, zbuf, acc, sid)
    _aggregate(D_H, kj, eb_hbm, out_hbm, src_v, dst_v, rows_v, xs, acc, sem, sem_s,
               cid, sid, wid)


def _spmm2_body(kj, p_hbm, eb_hbm, w_hbm, out_hbm,
                src_v, dst_v, rows_v, zbuf, pv, x2v, w_v, xs, acc, sem, sem_s):
    cid = lax.axis_index("c")
    sid = lax.axis_index("s")
    wid = sid * NC + cid
    base = sid * ROWS_PER_TILE

    pltpu.sync_copy(w_hbm, w_v)
    pltpu.sync_copy(p_hbm.at[0, pl.ds(base, ROWS_PER_TILE)], pv.at[0])
    pltpu.sync_copy(p_hbm.at[1, pl.ds(base, ROWS_PER_TILE)], pv.at[1])

    zero_i = jnp.zeros((L,), jnp.int32)
    riota = lax.iota(jnp.int32, L)
    wregs = [w_v[pl.ds(t * L, L)] for t in range(8 * D_H // L)]

    def _wscal(k, j):
        flat = k * 8 + j
        return wregs[flat // L][flat % L]

    def _group(g, carry):
        rows = g * L + riota
        cols = [None] * D_H
        for k in range(D_H):
            a = plsc.load_gather(pv, [zero_i, rows, zero_i + k])
            b = plsc.load_gather(pv, [zero_i + 1, rows, zero_i + k])
            cols[k] = jnp.maximum(a + b, 0.0)
        for j in range(8):
            o = cols[0] * _wscal(0, j)
            for k in range(1, D_H):
                o = o + cols[k] * _wscal(k, j)
            plsc.store_scatter(x2v, [rows, zero_i + j], o)
        return carry

    lax.fori_loop(0, ROWS_PER_TILE // L, _group, 0)
    pltpu.sync_copy(x2v, xs.at[pl.ds(base, ROWS_PER_TILE)])

    _zero_acc(8, zbuf, acc, sid)
    _aggregate(8, kj, eb_hbm, out_hbm, src_v, dst_v, rows_v, xs, acc, sem, sem_s,
               cid, sid, wid)


@functools.cache
def _build_spmm1(kj):
    mesh = plsc.VectorSubcoreMesh(
        core_axis_name="c", subcore_axis_name="s", num_cores=NC, num_subcores=NS
    )
    return pl.kernel(
        functools.partial(_spmm1_body, kj),
        out_type=jax.ShapeDtypeStruct((NC, N_PAD, D_H), jnp.float32),
        mesh=mesh,
        scratch_types=[
            pltpu.VMEM((kj, CHUNK), jnp.int32),
            pltpu.VMEM((kj, CHUNK), jnp.int32),
            pltpu.VMEM((4, CHUNK, D_H), jnp.float32),
            pltpu.VMEM((ROWS_PER_TILE, D_H), jnp.float32),
            pltpu.VMEM_SHARED((N_PAD, D_H), jnp.float32),
            pltpu.VMEM_SHARED((N_PAD, D_H), jnp.float32),
            pltpu.SemaphoreType.DMA,
            pltpu.SemaphoreType.DMA,
        ],
        compiler_params=pltpu.CompilerParams(use_tc_tiling_on_sc=False),
    )


@functools.cache
def _build_spmm2(kj):
    mesh = plsc.VectorSubcoreMesh(
        core_axis_name="c", subcore_axis_name="s", num_cores=NC, num_subcores=NS
    )
    return pl.kernel(
        functools.partial(_spmm2_body, kj),
        out_type=jax.ShapeDtypeStruct((NC, N_PAD, 8), jnp.float32),
        mesh=mesh,
        scratch_types=[
            pltpu.VMEM((kj, CHUNK), jnp.int32),
            pltpu.VMEM((kj, CHUNK), jnp.int32),
            pltpu.VMEM((4, CHUNK, 8), jnp.float32),
            pltpu.VMEM((ROWS_PER_TILE, 8), jnp.float32),
            pltpu.VMEM((2, ROWS_PER_TILE, D_H), jnp.float32),
            pltpu.VMEM((ROWS_PER_TILE, 8), jnp.float32),
            pltpu.VMEM((8 * D_H,), jnp.float32),
            pltpu.VMEM_SHARED((N_PAD, 8), jnp.float32),
            pltpu.VMEM_SHARED((N_PAD, 8), jnp.float32),
            pltpu.SemaphoreType.DMA,
            pltpu.SemaphoreType.DMA,
        ],
        compiler_params=pltpu.CompilerParams(
            use_tc_tiling_on_sc=False, needs_layout_passes=False
        ),
    )


ROWS_PER_WORKER = 400
N_COMBINE_W = N_NODES // ROWS_PER_WORKER


def _combine_sc_body(q_hbm, out_hbm, qv, sv):
    cid = lax.axis_index("c")
    sid = lax.axis_index("s")
    wid = sid * NC + cid

    @pl.when(wid < N_COMBINE_W)
    def _():
        base = wid * ROWS_PER_WORKER
        pltpu.sync_copy(q_hbm.at[0, pl.ds(base, ROWS_PER_WORKER)], qv.at[0])
        pltpu.sync_copy(q_hbm.at[1, pl.ds(base, ROWS_PER_WORKER)], qv.at[1])

        zero_i = jnp.zeros((L,), jnp.int32)
        lane = lax.iota(jnp.int32, L)

        @plsc.parallel_loop(0, ROWS_PER_WORKER * D_OUT // L, unroll=2)
        def _pack(t):
            fo = t * L + lane
            r = fo // D_OUT
            c = fo - r * D_OUT
            a = plsc.load_gather(qv, [zero_i, r, c])
            b = plsc.load_gather(qv, [zero_i + 1, r, c])
            plsc.store_scatter(sv, [r, c], a + b)

        pltpu.sync_copy(sv, out_hbm.at[pl.ds(base, ROWS_PER_WORKER)])


@functools.cache
def _build_combine_sc():
    mesh = plsc.VectorSubcoreMesh(
        core_axis_name="c", subcore_axis_name="s", num_cores=NC, num_subcores=NS
    )
    return pl.kernel(
        _combine_sc_body,
        out_type=jax.ShapeDtypeStruct((N_NODES, D_OUT), jnp.float32),
        mesh=mesh,
        scratch_types=[
            pltpu.VMEM((2, ROWS_PER_WORKER, 8), jnp.float32),
            pltpu.VMEM((ROWS_PER_WORKER, D_OUT), jnp.float32),
        ],
        compiler_params=pltpu.CompilerParams(
            use_tc_tiling_on_sc=False, needs_layout_passes=False
        ),
    )


def _mm1_body(f_ref, w1_ref, o_ref):
    o_ref[...] = jnp.dot(f_ref[...], w1_ref[...],
                         preferred_element_type=jnp.float32)


def kernel(features, edge_index, W1, W2):
    e = edge_index.shape[1]
    kj = (e + NW * CHUNK - 1) // (NW * CHUNK)
    e_pad = NW * CHUNK * kj
    eb = jnp.pad(
        edge_index.astype(jnp.int32),
        ((0, 0), (0, e_pad - e)),
        constant_values=N_NODES,
    ).reshape(2, NW, kj, CHUNK)

    w2f = jnp.pad(W2, ((0, 0), (0, 8 - D_OUT))).reshape(8 * D_H)

    x1 = pl.pallas_call(
        _mm1_body,
        grid=(2,),
        in_specs=[
            pl.BlockSpec((N_PAD // 2, D_IN), lambda i: (i, 0)),
            pl.BlockSpec((D_IN, D_H), lambda i: (0, 0)),
        ],
        out_specs=pl.BlockSpec((N_PAD // 2, D_H), lambda i: (i, 0)),
        out_shape=jax.ShapeDtypeStruct((N_PAD, D_H), jnp.float32),
    )(features, W1)

    p = _build_spmm1(kj)(x1, eb)

    q = _build_spmm2(kj)(p, eb, w2f)

    return _build_combine_sc()(q)


# speedup vs baseline: 1.0681x; 1.0522x over previous
import functools

import jax
import jax.numpy as jnp
from jax import lax
from jax.experimental import pallas as pl
from jax.experimental.pallas import tpu as pltpu
from jax.experimental.pallas import tpu_sc as plsc

N_NODES = 10000
D_IN = 128
D_H = 16
D_OUT = 7

NC = 2
NS = 16
NW = NC * NS

N_PAD = 10240
CHUNK = 128
ROWS_PER_TILE = N_PAD // NS
L = 16


def _aggregate(d, kj, eb_hbm, out_hbm, src_v, dst_v, rows_v, xs, acc, sem, sem_s,
               cid, sid, wid):
    pltpu.sync_copy(eb_hbm.at[0, wid], src_v)
    pltpu.sync_copy(eb_hbm.at[1, wid], dst_v)
    plsc.subcore_barrier()

    for jp in range(3):
        pltpu.async_copy(xs.at[src_v.at[jp]], rows_v.at[jp], sem)

    def _chunk(j, carry):
        buf = lax.rem(j, 6)
        pltpu.make_async_copy(xs.at[src_v.at[j]], rows_v.at[buf], sem).wait()

        @pl.when(j >= 3)
        def _():
            b2 = lax.rem(j + 3, 6)
            pltpu.make_async_copy(
                rows_v.at[b2], acc.at[dst_v.at[j - 3]], sem_s
            ).wait()

        @pl.when(j + 3 < kj)
        def _():
            pltpu.async_copy(
                xs.at[src_v.at[j + 3]], rows_v.at[lax.rem(j + 3, 6)], sem
            )

        pltpu.async_copy(rows_v.at[buf], acc.at[dst_v.at[j]], sem_s, add=True)
        return carry

    lax.fori_loop(0, kj, _chunk, 0, unroll=2)
    for j in (kj - 3, kj - 2, kj - 1):
        pltpu.make_async_copy(
            rows_v.at[lax.rem(j, 6)], acc.at[dst_v.at[j]], sem_s
        ).wait()
    plsc.subcore_barrier()

    pltpu.sync_copy(
        acc.at[pl.ds(sid * ROWS_PER_TILE, ROWS_PER_TILE)],
        out_hbm.at[cid, pl.ds(sid * ROWS_PER_TILE, ROWS_PER_TILE)],
    )


def _zero_acc(d, zbuf, acc, sid):
    if d == L:
        @plsc.parallel_loop(0, ROWS_PER_TILE, unroll=4)
        def _zero(i):
            zbuf[i, :] = jnp.zeros((L,), jnp.float32)
    else:
        zf = jnp.zeros((L,), jnp.float32)
        zi = jnp.zeros((L,), jnp.int32)
        riota = lax.iota(jnp.int32, L)

        @plsc.parallel_loop(0, ROWS_PER_TILE // L, unroll=2)
        def _zero(g):
            rows = g * L + riota
            for j in range(d):
                plsc.store_scatter(zbuf, [rows, zi + j], zf)
    pltpu.sync_copy(zbuf, acc.at[pl.ds(sid * ROWS_PER_TILE, ROWS_PER_TILE)])


def _spmm1_body(kj, x_hbm, eb_hbm, out_hbm,
                src_v, dst_v, rows_v, zbuf, xs, acc, sem, sem_s):
    cid = lax.axis_index("c")
    sid = lax.axis_index("s")
    wid = sid * NC + cid

    stage = pltpu.make_async_copy(
        x_hbm.at[pl.ds(sid * ROWS_PER_TILE, ROWS_PER_TILE)],
        xs.at[pl.ds(sid * ROWS_PER_TILE, ROWS_PER_TILE)],
        sem,
    )
    stage.start()
    _zero_acc(D_H, zbuf, acc, sid)
    stage.wait()
    _aggregate(D_H, kj, eb_hbm, out_hbm, src_v, dst_v, rows_v, xs, acc, sem, sem_s,
               cid, sid, wid)


def _spmm2_body(kj, p_hbm, eb_hbm, w_hbm, out_hbm,
                src_v, dst_v, rows_v, zbuf, pv, x2v, w_v, xs, acc, sem, sem_s):
    cid = lax.axis_index("c")
    sid = lax.axis_index("s")
    wid = sid * NC + cid
    base = sid * ROWS_PER_TILE

    st_w = pltpu.make_async_copy(w_hbm, w_v, sem)
    st_0 = pltpu.make_async_copy(p_hbm.at[0, pl.ds(base, ROWS_PER_TILE)], pv.at[0], sem)
    st_1 = pltpu.make_async_copy(p_hbm.at[1, pl.ds(base, ROWS_PER_TILE)], pv.at[1], sem)
    st_w.start()
    st_0.start()
    st_1.start()
    _zero_acc(8, zbuf, acc, sid)
    st_w.wait()
    st_0.wait()
    st_1.wait()

    zero_i = jnp.zeros((L,), jnp.int32)
    riota = lax.iota(jnp.int32, L)
    wregs = [w_v[pl.ds(t * L, L)] for t in range(8 * D_H // L)]

    def _wscal(k, j):
        flat = k * 8 + j
        return wregs[flat // L][flat % L]

    def _group(g, carry):
        rows = g * L + riota
        cols = [None] * D_H
        for k in range(D_H):
            a = plsc.load_gather(pv, [zero_i, rows, zero_i + k])
            b = plsc.load_gather(pv, [zero_i + 1, rows, zero_i + k])
            cols[k] = jnp.maximum(a + b, 0.0)
        for j in range(8):
            o = cols[0] * _wscal(0, j)
            for k in range(1, D_H):
                o = o + cols[k] * _wscal(k, j)
            plsc.store_scatter(x2v, [rows, zero_i + j], o)
        return carry

    lax.fori_loop(0, ROWS_PER_TILE // L, _group, 0)
    pltpu.sync_copy(x2v, xs.at[pl.ds(base, ROWS_PER_TILE)])

    _aggregate(8, kj, eb_hbm, out_hbm, src_v, dst_v, rows_v, xs, acc, sem, sem_s,
               cid, sid, wid)


@functools.cache
def _build_spmm1(kj):
    mesh = plsc.VectorSubcoreMesh(
        core_axis_name="c", subcore_axis_name="s", num_cores=NC, num_subcores=NS
    )
    return pl.kernel(
        functools.partial(_spmm1_body, kj),
        out_type=jax.ShapeDtypeStruct((NC, N_PAD, D_H), jnp.float32),
        mesh=mesh,
        scratch_types=[
            pltpu.VMEM((kj, CHUNK), jnp.int32),
            pltpu.VMEM((kj, CHUNK), jnp.int32),
            pltpu.VMEM((6, CHUNK, D_H), jnp.float32),
            pltpu.VMEM((ROWS_PER_TILE, D_H), jnp.float32),
            pltpu.VMEM_SHARED((N_PAD, D_H), jnp.float32),
            pltpu.VMEM_SHARED((N_PAD, D_H), jnp.float32),
            pltpu.SemaphoreType.DMA,
            pltpu.SemaphoreType.DMA,
        ],
        compiler_params=pltpu.CompilerParams(use_tc_tiling_on_sc=False),
    )


@functools.cache
def _build_spmm2(kj):
    mesh = plsc.VectorSubcoreMesh(
        core_axis_name="c", subcore_axis_name="s", num_cores=NC, num_subcores=NS
    )
    return pl.kernel(
        functools.partial(_spmm2_body, kj),
        out_type=jax.ShapeDtypeStruct((NC, N_PAD, 8), jnp.float32),
        mesh=mesh,
        scratch_types=[
            pltpu.VMEM((kj, CHUNK), jnp.int32),
            pltpu.VMEM((kj, CHUNK), jnp.int32),
            pltpu.VMEM((6, CHUNK, 8), jnp.float32),
            pltpu.VMEM((ROWS_PER_TILE, 8), jnp.float32),
            pltpu.VMEM((2, ROWS_PER_TILE, D_H), jnp.float32),
            pltpu.VMEM((ROWS_PER_TILE, 8), jnp.float32),
            pltpu.VMEM((8 * D_H,), jnp.float32),
            pltpu.VMEM_SHARED((N_PAD, 8), jnp.float32),
            pltpu.VMEM_SHARED((N_PAD, 8), jnp.float32),
            pltpu.SemaphoreType.DMA,
            pltpu.SemaphoreType.DMA,
        ],
        compiler_params=pltpu.CompilerParams(
            use_tc_tiling_on_sc=False, needs_layout_passes=False
        ),
    )


ROWS_PER_WORKER = 400
N_COMBINE_W = N_NODES // ROWS_PER_WORKER


def _combine_sc_body(q_hbm, out_hbm, qv, sv):
    cid = lax.axis_index("c")
    sid = lax.axis_index("s")
    wid = sid * NC + cid

    @pl.when(wid < N_COMBINE_W)
    def _():
        base = wid * ROWS_PER_WORKER
        pltpu.sync_copy(q_hbm.at[0, pl.ds(base, ROWS_PER_WORKER)], qv.at[0])
        pltpu.sync_copy(q_hbm.at[1, pl.ds(base, ROWS_PER_WORKER)], qv.at[1])

        zero_i = jnp.zeros((L,), jnp.int32)
        lane = lax.iota(jnp.int32, L)

        @plsc.parallel_loop(0, ROWS_PER_WORKER * D_OUT // L, unroll=2)
        def _pack(t):
            fo = t * L + lane
            r = fo // D_OUT
            c = fo - r * D_OUT
            a = plsc.load_gather(qv, [zero_i, r, c])
            b = plsc.load_gather(qv, [zero_i + 1, r, c])
            plsc.store_scatter(sv, [r, c], a + b)

        pltpu.sync_copy(sv, out_hbm.at[pl.ds(base, ROWS_PER_WORKER)])


@functools.cache
def _build_combine_sc():
    mesh = plsc.VectorSubcoreMesh(
        core_axis_name="c", subcore_axis_name="s", num_cores=NC, num_subcores=NS
    )
    return pl.kernel(
        _combine_sc_body,
        out_type=jax.ShapeDtypeStruct((N_NODES, D_OUT), jnp.float32),
        mesh=mesh,
        scratch_types=[
            pltpu.VMEM((2, ROWS_PER_WORKER, 8), jnp.float32),
            pltpu.VMEM((ROWS_PER_WORKER, D_OUT), jnp.float32),
        ],
        compiler_params=pltpu.CompilerParams(
            use_tc_tiling_on_sc=False, needs_layout_passes=False
        ),
    )


def _mm1_body(f_ref, w1_ref, o_ref):
    o_ref[...] = jnp.dot(f_ref[...], w1_ref[...],
                         preferred_element_type=jnp.float32)


def kernel(features, edge_index, W1, W2):
    e = edge_index.shape[1]
    kj = (e + NW * CHUNK - 1) // (NW * CHUNK)
    e_pad = NW * CHUNK * kj
    eb = jnp.pad(
        edge_index.astype(jnp.int32),
        ((0, 0), (0, e_pad - e)),
        constant_values=N_NODES,
    ).reshape(2, NW, kj, CHUNK)

    w2f = jnp.pad(W2, ((0, 0), (0, 8 - D_OUT))).reshape(8 * D_H)

    x1 = pl.pallas_call(
        _mm1_body,
        grid=(2,),
        in_specs=[
            pl.BlockSpec((N_PAD // 2, D_IN), lambda i: (i, 0)),
            pl.BlockSpec((D_IN, D_H), lambda i: (0, 0)),
        ],
        out_specs=pl.BlockSpec((N_PAD // 2, D_H), lambda i: (i, 0)),
        out_shape=jax.ShapeDtypeStruct((N_PAD, D_H), jnp.float32),
    )(features, W1)

    p = _build_spmm1(kj)(x1, eb)

    q = _build_spmm2(kj)(p, eb, w2f)

    return _build_combine_sc()(q)
